# Initial kernel scaffold; baseline (speedup 1.0000x reference)
import jax
import jax.numpy as jnp
from jax.experimental import pallas as pl


def kernel(adj, rewards, params):
    raise NotImplementedError("write your pallas kernel here")


# speedup vs baseline: 4.5558x; 4.5558x over previous
import jax
import jax.numpy as jnp
import numpy as np
from jax.experimental import pallas as pl
from jax.experimental.pallas import tpu as pltpu

N = 256
H = 256
B = 8
ATT_H = 128
L = 3
_NPAIR = (N - 2) * (N - 1) // 2
_NZERO = N * N - _NPAIR
_LN2 = float(np.log(2.0))


def _shift_up(x):
    return jnp.concatenate([x[1:, :], jnp.zeros((1, x.shape[1]), jnp.float32)], axis=0)


def _body(rew_ref, adj_ref, *refs):
    nparams = 12 * L + 6
    prefs = refs[:nparams]
    out_ref = refs[nparams]
    G_ref = refs[nparams + 1]
    Wm_ref = refs[nparams + 2]

    row = jax.lax.broadcasted_iota(jnp.int32, (N, N), 0)
    col = jax.lax.broadcasted_iota(jnp.int32, (N, N), 1)
    state = jnp.where(row == col, 1.0, 0.0).astype(jnp.float32)

    for l in range(L):
        (msg_w1, msg_b1, msg_w2, msg_b2,
         att_w1, att_b1, att_w2, att_b2,
         gru_wih, gru_bih, gru_whh, gru_bhh) = (r[...] for r in prefs[12 * l:12 * l + 12])

        wd_m, wu_m = msg_w1[:H, :], msg_w1[H:, :]
        wd_a, wu_a = att_w1[:H, :], att_w1[H:, :]

        sw = jnp.dot(state, wd_m, preferred_element_type=jnp.float32)
        msg_pre = (sw - _shift_up(sw)) + (_shift_up(wu_m) - wu_m) + msg_b1
        msg = jnp.dot(jnp.maximum(msg_pre, 0.0), msg_w2,
                      preferred_element_type=jnp.float32) + msg_b2

        sa = jnp.dot(state, wd_a, preferred_element_type=jnp.float32)
        att_pre = (sa - _shift_up(sa)) + (_shift_up(wu_a) - wu_a) + att_b1
        att = jax.nn.sigmoid(
            jnp.dot(jnp.maximum(att_pre, 0.0), att_w2,
                    preferred_element_type=jnp.float32) + att_b2)

        m = msg * att
        state_msg = jnp.concatenate([jnp.zeros((1, H), jnp.float32), m[:N - 1, :]], axis=0)

        gi = jnp.dot(state_msg, gru_wih, preferred_element_type=jnp.float32) + gru_bih
        gh = jnp.dot(state, gru_whh, preferred_element_type=jnp.float32) + gru_bhh
        r_g = jax.nn.sigmoid(gi[:, :H] + gh[:, :H])
        z_g = jax.nn.sigmoid(gi[:, H:2 * H] + gh[:, H:2 * H])
        n_g = jnp.tanh(gi[:, 2 * H:] + r_g * gh[:, 2 * H:])
        state = (1.0 - z_g) * n_g + z_g * state

    ow1, ob1, ow2, ob2, ow3r, ob3 = (r[...] for r in prefs[12 * L:])

    G_ref[...] = jnp.dot(state, ow1, preferred_element_type=jnp.float32)

    wm = jnp.zeros((N, N), jnp.float32)
    for b in range(B):
        wm = wm + rew_ref[0, b] * adj_ref[b]
    Wm_ref[...] = wm

    jj = jax.lax.broadcasted_iota(jnp.int32, (N, 1), 0)
    lane = jax.lax.broadcasted_iota(jnp.int32, (1, N), 1)

    def pair_step(i, carry):
        accA, accB = carry
        gi_row = G_ref[pl.ds(i, 1), :]
        h1 = jnp.maximum(gi_row - G_ref[...] + ob1, 0.0)
        h2 = jnp.maximum(jnp.dot(h1, ow2, preferred_element_type=jnp.float32) + ob2, 0.0)
        t = jnp.sum(h2 * ow3r, axis=1, keepdims=True) + ob3[0, 0]
        mask = jj >= i + 2
        sp = jnp.maximum(t, 0.0) + jnp.log1p(jnp.exp(-jnp.abs(t)))
        accA = accA + jnp.where(mask, sp, 0.0)
        tm = jnp.where(mask, t, 0.0)
        vrow = jnp.sum(Wm_ref[...] * tm, axis=0, keepdims=True)
        accB = accB + jnp.where(lane == i, vrow, 0.0)
        return accA, accB

    accA0 = jnp.zeros((N, 1), jnp.float32)
    accB0 = jnp.zeros((1, N), jnp.float32)
    accA, accB = jax.lax.fori_loop(0, N, pair_step, (accA0, accB0))

    acc1 = jnp.sum(accA)
    acc2 = jnp.sum(accB)

    s_rew = rew_ref[0, 0]
    for b in range(1, B):
        s_rew = s_rew + rew_ref[0, b]

    loss = (s_rew * (acc1 + _NZERO * _LN2) - acc2) * (1.0 / (N * N * B))
    out_ref[0, 0] = loss


def kernel(adj, rewards, params):
    adj3 = adj.reshape(B, N, N).astype(jnp.float32)
    rew = rewards.reshape(1, B).astype(jnp.float32)

    flat = []
    for lp in params['layers']:
        flat += [lp['msg_w1'], lp['msg_b1'].reshape(1, -1),
                 lp['msg_w2'], lp['msg_b2'].reshape(1, -1),
                 lp['att_w1'], lp['att_b1'].reshape(1, -1),
                 lp['att_w2'], lp['att_b2'].reshape(1, -1),
                 lp['gru_wih'], lp['gru_bih'].reshape(1, -1),
                 lp['gru_whh'], lp['gru_bhh'].reshape(1, -1)]
    flat += [params['out_w1'], params['out_b1'].reshape(1, -1),
             params['out_w2'], params['out_b2'].reshape(1, -1),
             params['out_w3'].reshape(1, -1), params['out_b3'].reshape(1, 1)]

    in_specs = ([pl.BlockSpec(memory_space=pltpu.SMEM),
                 pl.BlockSpec(memory_space=pltpu.VMEM)] +
                [pl.BlockSpec(memory_space=pltpu.VMEM)] * len(flat))

    out = pl.pallas_call(
        _body,
        out_shape=jax.ShapeDtypeStruct((1, 1), jnp.float32),
        in_specs=in_specs,
        out_specs=pl.BlockSpec(memory_space=pltpu.SMEM),
        scratch_shapes=[pltpu.VMEM((N, N), jnp.float32),
                        pltpu.VMEM((N, N), jnp.float32)],
    )(rew, adj3, *flat)
    return out[0, 0]


# speedup vs baseline: 6.1597x; 1.3521x over previous
import jax
import jax.numpy as jnp
import numpy as np
from jax.experimental import pallas as pl
from jax.experimental.pallas import tpu as pltpu

N = 256
H = 256
B = 8
ATT_H = 128
L = 3
_NPAIR = (N - 2) * (N - 1) // 2
_NZERO = N * N - _NPAIR
_LN2 = float(np.log(2.0))


def _shift_up(x):
    return jnp.concatenate([x[1:, :], jnp.zeros((1, x.shape[1]), jnp.float32)], axis=0)


def _body(rew_ref, adj_ref, *refs):
    nparams = 12 * L + 6
    prefs = refs[:nparams]
    out_ref = refs[nparams]
    G_ref = refs[nparams + 1]
    Wm_ref = refs[nparams + 2]

    row = jax.lax.broadcasted_iota(jnp.int32, (N, N), 0)
    col = jax.lax.broadcasted_iota(jnp.int32, (N, N), 1)
    state = jnp.where(row == col, 1.0, 0.0).astype(jnp.float32)

    for l in range(L):
        (msg_w1, msg_b1, msg_w2, msg_b2,
         att_w1, att_b1, att_w2, att_b2,
         gru_wih, gru_bih, gru_whh, gru_bhh) = (r[...] for r in prefs[12 * l:12 * l + 12])

        wd_m, wu_m = msg_w1[:H, :], msg_w1[H:, :]
        wd_a, wu_a = att_w1[:H, :], att_w1[H:, :]

        sw = jnp.dot(state, wd_m, preferred_element_type=jnp.float32)
        msg_pre = (sw - _shift_up(sw)) + (_shift_up(wu_m) - wu_m) + msg_b1
        msg = jnp.dot(jnp.maximum(msg_pre, 0.0), msg_w2,
                      preferred_element_type=jnp.float32) + msg_b2

        sa = jnp.dot(state, wd_a, preferred_element_type=jnp.float32)
        att_pre = (sa - _shift_up(sa)) + (_shift_up(wu_a) - wu_a) + att_b1
        att = jax.nn.sigmoid(
            jnp.dot(jnp.maximum(att_pre, 0.0), att_w2,
                    preferred_element_type=jnp.float32) + att_b2)

        m = msg * att
        state_msg = jnp.concatenate([jnp.zeros((1, H), jnp.float32), m[:N - 1, :]], axis=0)

        gi = jnp.dot(state_msg, gru_wih, preferred_element_type=jnp.float32) + gru_bih
        gh = jnp.dot(state, gru_whh, preferred_element_type=jnp.float32) + gru_bhh
        r_g = jax.nn.sigmoid(gi[:, :H] + gh[:, :H])
        z_g = jax.nn.sigmoid(gi[:, H:2 * H] + gh[:, H:2 * H])
        n_g = jnp.tanh(gi[:, 2 * H:] + r_g * gh[:, 2 * H:])
        state = (1.0 - z_g) * n_g + z_g * state

    ow1, ob1, ow2, ob2, ow3r, ob3 = (r[...] for r in prefs[12 * L:])

    G_ref[...] = jnp.dot(state, ow1, preferred_element_type=jnp.float32)

    wm = jnp.zeros((N, N), jnp.float32)
    for b in range(B):
        wm = wm + rew_ref[0, b] * adj_ref[b]
    Wm_ref[...] = wm

    jj = jax.lax.broadcasted_iota(jnp.int32, (N, 1), 0)
    lane = jax.lax.broadcasted_iota(jnp.int32, (1, N), 1)

    def pair_step(i, carry):
        accA, accB = carry
        q = 253 - i
        gi_row = G_ref[pl.ds(i, 1), :]
        gq_row = G_ref[pl.ds(q, 1), :]
        G = G_ref[...]
        gshift = pltpu.roll(G, -(i + 2), axis=0)
        part1 = jj < 254 - i
        src = jnp.where(part1, gshift, G)
        top = jnp.where(part1, gi_row, gq_row)
        h1 = jnp.maximum(top - src + ob1, 0.0)
        h2 = jnp.maximum(jnp.dot(h1, ow2, preferred_element_type=jnp.float32) + ob2, 0.0)
        t = jnp.sum(h2 * ow3r, axis=1, keepdims=True) + ob3[0, 0]
        valid = jj != 254 - i
        sp = jnp.maximum(t, 0.0) + jnp.log1p(jnp.exp(-jnp.abs(t)))
        accA = accA + jnp.where(valid, sp, 0.0)
        tm = jnp.where(valid, t, 0.0)
        tm1 = jnp.where(part1, tm, 0.0)
        tm2 = tm - tm1
        tm1u = pltpu.roll(tm1, i + 2, axis=0)
        vrow1 = jnp.sum(Wm_ref[...] * tm1u, axis=0, keepdims=True)
        vrow2 = jnp.sum(Wm_ref[...] * tm2, axis=0, keepdims=True)
        accB = accB + jnp.where(lane == i, vrow1, 0.0) + jnp.where(lane == q, vrow2, 0.0)
        return accA, accB

    accA0 = jnp.zeros((N, 1), jnp.float32)
    accB0 = jnp.zeros((1, N), jnp.float32)
    accA, accB = jax.lax.fori_loop(0, 127, pair_step, (accA0, accB0))

    acc1 = jnp.sum(accA)
    acc2 = jnp.sum(accB)

    s_rew = rew_ref[0, 0]
    for b in range(1, B):
        s_rew = s_rew + rew_ref[0, b]

    loss = (s_rew * (acc1 + _NZERO * _LN2) - acc2) * (1.0 / (N * N * B))
    out_ref[0, 0] = loss


def kernel(adj, rewards, params):
    adj3 = adj.reshape(B, N, N).astype(jnp.float32)
    rew = rewards.reshape(1, B).astype(jnp.float32)

    flat = []
    for lp in params['layers']:
        flat += [lp['msg_w1'], lp['msg_b1'].reshape(1, -1),
                 lp['msg_w2'], lp['msg_b2'].reshape(1, -1),
                 lp['att_w1'], lp['att_b1'].reshape(1, -1),
                 lp['att_w2'], lp['att_b2'].reshape(1, -1),
                 lp['gru_wih'], lp['gru_bih'].reshape(1, -1),
                 lp['gru_whh'], lp['gru_bhh'].reshape(1, -1)]
    flat += [params['out_w1'], params['out_b1'].reshape(1, -1),
             params['out_w2'], params['out_b2'].reshape(1, -1),
             params['out_w3'].reshape(1, -1), params['out_b3'].reshape(1, 1)]

    in_specs = ([pl.BlockSpec(memory_space=pltpu.SMEM),
                 pl.BlockSpec(memory_space=pltpu.VMEM)] +
                [pl.BlockSpec(memory_space=pltpu.VMEM)] * len(flat))

    out = pl.pallas_call(
        _body,
        out_shape=jax.ShapeDtypeStruct((1, 1), jnp.float32),
        in_specs=in_specs,
        out_specs=pl.BlockSpec(memory_space=pltpu.SMEM),
        scratch_shapes=[pltpu.VMEM((N, N), jnp.float32),
                        pltpu.VMEM((N, N), jnp.float32)],
    )(rew, adj3, *flat)
    return out[0, 0]
